# Initial kernel scaffold; baseline (speedup 1.0000x reference)
import jax
import jax.numpy as jnp
from jax.experimental import pallas as pl


def kernel(x, edge_index, batch, W0_1, b0_1, W0_2, b0_2, W1_1, b1_1, W1_2, b1_2, w):
    raise NotImplementedError("write your pallas kernel here")


# speedup vs baseline: 5.0054x; 5.0054x over previous
import functools

import jax
import jax.numpy as jnp
from jax import lax
from jax.experimental import pallas as pl
from jax.experimental.pallas import tpu as pltpu
from jax.experimental.pallas import tpu_sc as plsc

N_NODES = 10000
N_EDGES = 320000
N_GRAPHS = 64
D = 128

NW = 32
EPT = N_EDGES // NW
CHUNK = 128
NCH = (EPT + CHUNK - 1) // CHUNK
EPT_PAD = NCH * CHUNK
ACC_R = 10112
STRIPE = ACC_R // 16
DUMMY = N_NODES


def _sc_segment_sum(src3d, dst3d, h):
    mesh = plsc.VectorSubcoreMesh(core_axis_name="c", subcore_axis_name="s")

    @functools.partial(
        pl.kernel,
        mesh=mesh,
        out_type=jax.ShapeDtypeStruct((2 * ACC_R, D), jnp.float32),
        scratch_types=[
            pltpu.VMEM((NCH, CHUNK), jnp.int32),
            pltpu.VMEM((NCH, CHUNK), jnp.int32),
            pltpu.VMEM((CHUNK, D), jnp.float32),
            pltpu.VMEM_SHARED((ACC_R, D), jnp.float32),
            pltpu.SemaphoreType.DMA,
        ],
    )
    def seg_kernel(src_hbm, dst_hbm, h_hbm, out_hbm, src_v, dst_v, stage, acc, sem):
        c = lax.axis_index("c")
        s = lax.axis_index("s")
        g = c * 16 + s

        pltpu.sync_copy(src_hbm.at[g], src_v)
        pltpu.sync_copy(dst_hbm.at[g], dst_v)

        def zrow(r, _):
            for k in range(D // 16):
                stage[r, pl.ds(16 * k, 16)] = jnp.zeros((16,), jnp.float32)
            return 0
        lax.fori_loop(0, CHUNK, zrow, 0)

        base = s * STRIPE
        def zc(i, _):
            pltpu.sync_copy(stage, acc.at[pl.ds(base + i * CHUNK, CHUNK)])
            return 0
        lax.fori_loop(0, STRIPE // CHUNK, zc, 0)
        rem = STRIPE % CHUNK
        if rem:
            pltpu.sync_copy(stage.at[pl.ds(0, rem)],
                            acc.at[pl.ds(base + (STRIPE // CHUNK) * CHUNK, rem)])
        plsc.subcore_barrier()

        def body(j, _):
            pltpu.async_copy(h_hbm.at[src_v.at[j]], stage, sem).wait()
            pltpu.sync_copy(stage, acc.at[dst_v.at[j]], add=True)
            return 0
        lax.fori_loop(0, NCH, body, 0)
        plsc.subcore_barrier()

        out_base = c * ACC_R + base
        def ec(i, _):
            pltpu.sync_copy(acc.at[pl.ds(base + i * CHUNK, CHUNK)], stage)
            pltpu.sync_copy(stage, out_hbm.at[pl.ds(out_base + i * CHUNK, CHUNK)])
            return 0
        lax.fori_loop(0, STRIPE // CHUNK, ec, 0)
        if rem:
            off = (STRIPE // CHUNK) * CHUNK
            pltpu.sync_copy(acc.at[pl.ds(base + off, rem)], stage.at[pl.ds(0, rem)])
            pltpu.sync_copy(stage.at[pl.ds(0, rem)],
                            out_hbm.at[pl.ds(out_base + off, rem)])

    return seg_kernel(src3d, dst3d, h)


def _mlp_body(p_ref, h_ref, wa_ref, ba_ref, wb_ref, bb_ref, o_ref):
    z = (p_ref[0:N_NODES, :] + p_ref[ACC_R:ACC_R + N_NODES, :] + h_ref[...])
    z = jnp.maximum(
        jnp.dot(z, wa_ref[...], preferred_element_type=jnp.float32) + ba_ref[...],
        0.0)
    z = jnp.dot(z, wb_ref[...], preferred_element_type=jnp.float32) + bb_ref[...]
    o_ref[...] = jnp.maximum(z, 0.0)


def _final_body(p_ref, h_ref, wa_ref, ba_ref, wb_ref, bb_ref, w_ref,
                bcol_ref, brow_ref, permoh_ref, o_ref):
    z = (p_ref[0:N_NODES, :] + p_ref[ACC_R:ACC_R + N_NODES, :] + h_ref[...])
    z = jnp.maximum(
        jnp.dot(z, wa_ref[...], preferred_element_type=jnp.float32) + ba_ref[...],
        0.0)
    emb = jnp.dot(z, wb_ref[...], preferred_element_type=jnp.float32) + bb_ref[...]

    iota_n = lax.broadcasted_iota(jnp.int32, (N_NODES, N_GRAPHS), 1)
    oh = (bcol_ref[...] == iota_n).astype(jnp.float32)
    iota_g = lax.broadcasted_iota(jnp.int32, (N_GRAPHS, N_NODES), 0)
    oh_t = (brow_ref[...] == iota_g).astype(jnp.float32)

    sums = jnp.dot(oh_t, emb, preferred_element_type=jnp.float32)
    cnts = jnp.sum(oh_t, axis=1, keepdims=True)
    mean = sums / jnp.maximum(cnts, 1.0)
    summary = 1.0 / (1.0 + jnp.exp(-mean))
    neg_summary = jnp.dot(permoh_ref[...], summary,
                          preferred_element_type=jnp.float32)

    zw = jnp.dot(emb, w_ref[...], preferred_element_type=jnp.float32)
    s_b = jnp.dot(oh, summary, preferred_element_type=jnp.float32)
    ns_b = jnp.dot(oh, neg_summary, preferred_element_type=jnp.float32)

    pos = 1.0 / (1.0 + jnp.exp(-jnp.sum(zw * s_b, axis=1, keepdims=True)))
    neg = 1.0 / (1.0 + jnp.exp(-jnp.sum(zw * ns_b, axis=1, keepdims=True)))
    pos_sum = jnp.sum(jnp.log(jnp.maximum(pos + 1e-15, 1e-15)))
    neg_sum = jnp.sum(jnp.log(jnp.maximum(1.0 - neg + 1e-15, 1e-15)))
    total = -(pos_sum + neg_sum) / N_NODES
    o_ref[...] = jnp.broadcast_to(total, (1, 1))


def kernel(x, edge_index, batch, W0_1, b0_1, W0_2, b0_2, W1_1, b1_1, W1_2, b1_2, w):
    src = edge_index[0].reshape(NW, EPT)
    dst = edge_index[1].reshape(NW, EPT)
    pad = ((0, 0), (0, EPT_PAD - EPT))
    src3d = jnp.pad(src, pad, constant_values=0).reshape(NW, NCH, CHUNK)
    dst3d = jnp.pad(dst, pad, constant_values=DUMMY).reshape(NW, NCH, CHUNK)

    b0_1r, b0_2r = b0_1.reshape(1, D), b0_2.reshape(1, D)
    b1_1r, b1_2r = b1_1.reshape(1, D), b1_2.reshape(1, D)
    bcol = batch.reshape(N_NODES, 1)
    brow = batch.reshape(1, N_NODES)
    perm = jax.random.permutation(jax.random.key(1), N_GRAPHS)
    perm_oh = jax.nn.one_hot(perm, N_GRAPHS, dtype=jnp.float32)

    p0 = _sc_segment_sum(src3d, dst3d, x)
    h1 = pl.pallas_call(
        _mlp_body,
        out_shape=jax.ShapeDtypeStruct((N_NODES, D), jnp.float32),
    )(p0, x, W0_1, b0_1r, W0_2, b0_2r)

    p1 = _sc_segment_sum(src3d, dst3d, h1)
    loss = pl.pallas_call(
        _final_body,
        out_shape=jax.ShapeDtypeStruct((1, 1), jnp.float32),
    )(p1, h1, W1_1, b1_1r, W1_2, b1_2r, w, bcol, brow, perm_oh)
    return loss[0, 0]


# speedup vs baseline: 5.6451x; 1.1278x over previous
import functools

import jax
import jax.numpy as jnp
from jax import lax
from jax.experimental import pallas as pl
from jax.experimental.pallas import tpu as pltpu
from jax.experimental.pallas import tpu_sc as plsc

N_NODES = 10000
N_EDGES = 320000
N_GRAPHS = 64
D = 128
DH = D // 2

EPS = N_EDGES // 16
CHUNK = 128
NBUF = 4
NCH = EPS // CHUNK + (EPS % CHUNK > 0)
NCH += (-NCH) % NBUF
EPT_PAD = NCH * CHUNK
ACC_R = 10112
STRIPE = ACC_R // 16
DUMMY = N_NODES


def _sc_segment_sum(src3d, dst3d, h_split):
    mesh = plsc.VectorSubcoreMesh(core_axis_name="c", subcore_axis_name="s")

    @functools.partial(
        pl.kernel,
        mesh=mesh,
        compiler_params=pltpu.CompilerParams(use_tc_tiling_on_sc=False),
        out_type=jax.ShapeDtypeStruct((2 * ACC_R, DH), jnp.float32),
        scratch_types=[
            pltpu.VMEM((NCH, CHUNK), jnp.int32),
            pltpu.VMEM((NCH, CHUNK), jnp.int32),
            *[pltpu.VMEM((CHUNK, DH), jnp.float32) for _ in range(NBUF)],
            pltpu.VMEM_SHARED((ACC_R, DH), jnp.float32),
            *[pltpu.SemaphoreType.DMA for _ in range(NBUF)],
        ],
    )
    def seg_kernel(src_hbm, dst_hbm, h_hbm, out_hbm, src_v, dst_v, *rest):
        stages = rest[:NBUF]
        acc = rest[NBUF]
        sems = rest[NBUF + 1:]
        stage = stages[0]
        c = lax.axis_index("c")
        s = lax.axis_index("s")

        pltpu.sync_copy(src_hbm.at[s], src_v)
        pltpu.sync_copy(dst_hbm.at[s], dst_v)
        coff = jnp.full((16,), c * N_NODES, dtype=jnp.int32)

        def shift(j, _):
            for k in range(CHUNK // 16):
                sl = pl.ds(16 * k, 16)
                src_v[j, sl] = src_v[j, sl] + coff
            return 0
        lax.fori_loop(0, NCH, shift, 0)

        def zrow(r, _):
            for k in range(DH // 16):
                stage[r, pl.ds(16 * k, 16)] = jnp.zeros((16,), jnp.float32)
            return 0
        lax.fori_loop(0, CHUNK, zrow, 0)

        base = s * STRIPE
        def zc(i, _):
            pltpu.sync_copy(stage, acc.at[pl.ds(base + i * CHUNK, CHUNK)])
            return 0
        lax.fori_loop(0, STRIPE // CHUNK, zc, 0)
        rem = STRIPE % CHUNK
        if rem:
            pltpu.sync_copy(stage.at[pl.ds(0, rem)],
                            acc.at[pl.ds(base + (STRIPE // CHUNK) * CHUNK, rem)])
        plsc.subcore_barrier()

        for b in range(NBUF):
            pltpu.async_copy(h_hbm.at[src_v.at[b]], stages[b], sems[b])

        def group(g, _):
            for b in range(NBUF):
                j = g * NBUF + b
                pltpu.make_async_copy(h_hbm.at[src_v.at[j]], stages[b],
                                      sems[b]).wait()
                pltpu.sync_copy(stages[b], acc.at[dst_v.at[j]], add=True)
                nxt = j + NBUF

                @pl.when(nxt < NCH)
                def _():
                    pltpu.async_copy(h_hbm.at[src_v.at[nxt]], stages[b],
                                     sems[b])
            return 0
        lax.fori_loop(0, NCH // NBUF, group, 0)
        plsc.subcore_barrier()

        out_base = c * ACC_R + base
        def ec(i, _):
            pltpu.sync_copy(acc.at[pl.ds(base + i * CHUNK, CHUNK)], stage)
            pltpu.sync_copy(stage, out_hbm.at[pl.ds(out_base + i * CHUNK, CHUNK)])
            return 0
        lax.fori_loop(0, STRIPE // CHUNK, ec, 0)
        if rem:
            off = (STRIPE // CHUNK) * CHUNK
            pltpu.sync_copy(acc.at[pl.ds(base + off, rem)], stage.at[pl.ds(0, rem)])
            pltpu.sync_copy(stage.at[pl.ds(0, rem)],
                            out_hbm.at[pl.ds(out_base + off, rem)])

    return seg_kernel(src3d, dst3d, h_split)


def _cat(p_ref, h_ref):
    agg = jnp.concatenate(
        [p_ref[0:N_NODES, :], p_ref[ACC_R:ACC_R + N_NODES, :]], axis=1)
    h = jnp.concatenate(
        [h_ref[0:N_NODES, :], h_ref[N_NODES:2 * N_NODES, :]], axis=1)
    return agg + h


def _mlp_body(p_ref, h_ref, wa_ref, ba_ref, wb_ref, bb_ref, o_ref):
    z = _cat(p_ref, h_ref)
    z = jnp.maximum(
        jnp.dot(z, wa_ref[...], preferred_element_type=jnp.float32) + ba_ref[...],
        0.0)
    z = jnp.dot(z, wb_ref[...], preferred_element_type=jnp.float32) + bb_ref[...]
    z = jnp.maximum(z, 0.0)
    o_ref[0:N_NODES, :] = z[:, 0:DH]
    o_ref[N_NODES:2 * N_NODES, :] = z[:, DH:D]


def _final_body(p_ref, h_ref, wa_ref, ba_ref, wb_ref, bb_ref, w_ref,
                bcol_ref, brow_ref, permoh_ref, o_ref):
    z = _cat(p_ref, h_ref)
    z = jnp.maximum(
        jnp.dot(z, wa_ref[...], preferred_element_type=jnp.float32) + ba_ref[...],
        0.0)
    emb = jnp.dot(z, wb_ref[...], preferred_element_type=jnp.float32) + bb_ref[...]

    iota_n = lax.broadcasted_iota(jnp.int32, (N_NODES, N_GRAPHS), 1)
    oh = (bcol_ref[...] == iota_n).astype(jnp.float32)
    iota_g = lax.broadcasted_iota(jnp.int32, (N_GRAPHS, N_NODES), 0)
    oh_t = (brow_ref[...] == iota_g).astype(jnp.float32)

    sums = jnp.dot(oh_t, emb, preferred_element_type=jnp.float32)
    cnts = jnp.sum(oh_t, axis=1, keepdims=True)
    mean = sums / jnp.maximum(cnts, 1.0)
    summary = 1.0 / (1.0 + jnp.exp(-mean))
    neg_summary = jnp.dot(permoh_ref[...], summary,
                          preferred_element_type=jnp.float32)

    zw = jnp.dot(emb, w_ref[...], preferred_element_type=jnp.float32)
    s_b = jnp.dot(oh, summary, preferred_element_type=jnp.float32)
    ns_b = jnp.dot(oh, neg_summary, preferred_element_type=jnp.float32)

    pos = 1.0 / (1.0 + jnp.exp(-jnp.sum(zw * s_b, axis=1, keepdims=True)))
    neg = 1.0 / (1.0 + jnp.exp(-jnp.sum(zw * ns_b, axis=1, keepdims=True)))
    pos_sum = jnp.sum(jnp.log(jnp.maximum(pos + 1e-15, 1e-15)))
    neg_sum = jnp.sum(jnp.log(jnp.maximum(1.0 - neg + 1e-15, 1e-15)))
    total = -(pos_sum + neg_sum) / N_NODES
    o_ref[...] = jnp.broadcast_to(total, (1, 1))


def kernel(x, edge_index, batch, W0_1, b0_1, W0_2, b0_2, W1_1, b1_1, W1_2, b1_2, w):
    src = edge_index[0].reshape(16, EPS)
    dst = edge_index[1].reshape(16, EPS)
    pad = ((0, 0), (0, EPT_PAD - EPS))
    src3d = jnp.pad(src, pad, constant_values=0).reshape(16, NCH, CHUNK)
    dst3d = jnp.pad(dst, pad, constant_values=DUMMY).reshape(16, NCH, CHUNK)

    x_split = jnp.concatenate([x[:, 0:DH], x[:, DH:D]], axis=0)
    b0_1r, b0_2r = b0_1.reshape(1, D), b0_2.reshape(1, D)
    b1_1r, b1_2r = b1_1.reshape(1, D), b1_2.reshape(1, D)
    bcol = batch.reshape(N_NODES, 1)
    brow = batch.reshape(1, N_NODES)
    perm = jax.random.permutation(jax.random.key(1), N_GRAPHS)
    perm_oh = jax.nn.one_hot(perm, N_GRAPHS, dtype=jnp.float32)

    p0 = _sc_segment_sum(src3d, dst3d, x_split)
    h1s = pl.pallas_call(
        _mlp_body,
        out_shape=jax.ShapeDtypeStruct((2 * N_NODES, DH), jnp.float32),
    )(p0, x_split, W0_1, b0_1r, W0_2, b0_2r)

    p1 = _sc_segment_sum(src3d, dst3d, h1s)
    loss = pl.pallas_call(
        _final_body,
        out_shape=jax.ShapeDtypeStruct((1, 1), jnp.float32),
    )(p1, h1s, W1_1, b1_1r, W1_2, b1_2r, w, bcol, brow, perm_oh)
    return loss[0, 0]
